# Initial kernel scaffold; baseline (speedup 1.0000x reference)
import jax
import jax.numpy as jnp
from jax.experimental import pallas as pl


def kernel(indices, tables):
    raise NotImplementedError("write your pallas kernel here")


# speedup vs baseline: 6.9925x; 6.9925x over previous
import functools

import jax
import jax.numpy as jnp
from jax import lax
from jax.experimental import pallas as pl
from jax.experimental.pallas import tpu as pltpu
from jax.experimental.pallas import tpu_sc as plsc

F = 26
V = 100000
D = 32
B = 4096
L = 20

NC = 2
NS = 16
NW = NC * NS

BAGS_PER_W = B // NW
CB = 64
N_HALF = BAGS_PER_W // CB
N_IDX = CB * L
N_GATHER = N_IDX // 128


def _make_kernel():
  mesh = plsc.VectorSubcoreMesh(
      core_axis_name="c", subcore_axis_name="s",
      num_cores=NC, num_subcores=NS)

  @functools.partial(
      pl.kernel,
      out_type=jax.ShapeDtypeStruct((B, F * D), jnp.float32),
      mesh=mesh,
      compiler_params=pltpu.CompilerParams(use_tc_tiling_on_sc=False),
      scratch_types=[
          pltpu.VMEM((N_IDX,), jnp.int32),
          pltpu.VMEM((N_IDX, D), jnp.float32),
          pltpu.VMEM((CB, D), jnp.float32),
          pltpu.SemaphoreType.DMA,
      ],
  )
  def kern(tab_hbm, idx_hbm, out_hbm, idx_v, rows_v, pooled_v, sem):
    wid = lax.axis_index("s") * NC + lax.axis_index("c")
    b0w = wid * BAGS_PER_W

    def step(t, carry):
      f = t // N_HALF
      half = t % N_HALF
      b0 = b0w + half * CB

      src_off = pl.multiple_of(f * (B * L) + b0 * L, N_IDX)
      pltpu.sync_copy(idx_hbm.at[pl.ds(src_off, N_IDX)], idx_v)

      off = f * V

      def addoff(j, c):
        idx_v[pl.ds(j * 16, 16)] = idx_v[pl.ds(j * 16, 16)] + off
        return c

      lax.fori_loop(0, N_IDX // 16, addoff, 0)

      copies = [
          pltpu.async_copy(
              tab_hbm.at[idx_v.at[pl.ds(j * 128, 128)]],
              rows_v.at[pl.ds(j * 128, 128)],
              sem,
          )
          for j in range(N_GATHER)
      ]
      for cp in copies:
        cp.wait()

      def bag(c, carry2):
        base = c * L
        a0 = rows_v[base, pl.ds(0, 16)]
        a1 = rows_v[base, pl.ds(16, 16)]
        for l in range(1, L):
          a0 = a0 + rows_v[base + l, pl.ds(0, 16)]
          a1 = a1 + rows_v[base + l, pl.ds(16, 16)]
        pooled_v[c, pl.ds(0, 16)] = a0
        pooled_v[c, pl.ds(16, 16)] = a1
        return carry2

      lax.fori_loop(0, CB, bag, 0)

      pltpu.sync_copy(pooled_v, out_hbm.at[pl.ds(b0, CB), pl.ds(f * D, D)])
      return carry

    lax.fori_loop(0, F * N_HALF, step, 0)

  return kern


_kern = _make_kernel()


@jax.jit
def kernel(indices, tables):
  idx_flat = indices.reshape(F * B * L).astype(jnp.int32)
  tab_flat = tables.reshape(F * V, D)
  out = _kern(tab_flat, idx_flat)
  return out.reshape(B, F, D)


# speedup vs baseline: 7.6081x; 1.0880x over previous
import functools

import jax
import jax.numpy as jnp
from jax import lax
from jax.experimental import pallas as pl
from jax.experimental.pallas import tpu as pltpu
from jax.experimental.pallas import tpu_sc as plsc

F = 26
V = 100000
D = 32
B = 4096
L = 20

NC = 2
NS = 16
NW = NC * NS

BAGS_PER_W = B // NW
CB = 32
NQ = BAGS_PER_W // CB
NSTEP = NQ * F
N_IDX = CB * L
N_GATHER = N_IDX // 128


def _make_kernel():
  mesh = plsc.VectorSubcoreMesh(
      core_axis_name="c", subcore_axis_name="s",
      num_cores=NC, num_subcores=NS)

  @functools.partial(
      pl.kernel,
      out_type=jax.ShapeDtypeStruct((B, F * D), jnp.float32),
      mesh=mesh,
      compiler_params=pltpu.CompilerParams(use_tc_tiling_on_sc=False),
      scratch_types=[
          [pltpu.VMEM((N_IDX,), jnp.int32) for _ in range(4)],
          [pltpu.VMEM((N_IDX, D), jnp.float32) for _ in range(2)],
          pltpu.VMEM((CB, F * D), jnp.float32),
          [pltpu.SemaphoreType.DMA for _ in range(4)],
          [pltpu.SemaphoreType.DMA for _ in range(2)],
      ],
  )
  def kern(tab_hbm, idx_hbm, out_hbm, idx_bufs, rows_bufs, stage_v,
           idx_sems, row_sems):
    wid = lax.axis_index("s") * NC + lax.axis_index("c")
    b0w = wid * BAGS_PER_W

    def idx_src(t):
      f = t % F
      q = t // F
      off = f * (B * L) + (b0w + q * CB) * L
      return idx_hbm.at[pl.ds(pl.multiple_of(off, N_IDX), N_IDX)]

    def fire_idx(t, k):
      pltpu.async_copy(idx_src(t), idx_bufs[k], idx_sems[k])

    def drain_idx(k):
      pltpu.make_async_copy(
          idx_hbm.at[pl.ds(0, N_IDX)], idx_bufs[k], idx_sems[k]).wait()

    def addoff(t, k):
      off = (t % F) * V
      buf = idx_bufs[k]

      def body(j, c):
        buf[pl.ds(j * 16, 16)] = buf[pl.ds(j * 16, 16)] + off
        return c

      lax.fori_loop(0, N_IDX // 16, body, 0)

    def fire_gathers(k, r):
      for g in range(N_GATHER):
        pltpu.async_copy(
            tab_hbm.at[idx_bufs[k].at[pl.ds(g * 128, 128)]],
            rows_bufs[r].at[pl.ds(g * 128, 128)],
            row_sems[r])

    def drain_gathers(r):
      for g in range(N_GATHER):
        pltpu.make_async_copy(
            tab_hbm.at[pl.ds(0, 128)],
            rows_bufs[r].at[pl.ds(g * 128, 128)],
            row_sems[r]).wait()

    def reduce_step(t, r):
      f = t % F
      rows = rows_bufs[r]
      col = f * D

      def bag(c, carry):
        base = c * L
        p0 = [rows[base + l, pl.ds(0, 16)] for l in range(4)]
        p1 = [rows[base + l, pl.ds(16, 16)] for l in range(4)]
        for l in range(4, L):
          p0[l % 4] = p0[l % 4] + rows[base + l, pl.ds(0, 16)]
          p1[l % 4] = p1[l % 4] + rows[base + l, pl.ds(16, 16)]
        stage_v[c, pl.ds(col, 16)] = (p0[0] + p0[1]) + (p0[2] + p0[3])
        stage_v[c, pl.ds(col + 16, 16)] = (p1[0] + p1[1]) + (p1[2] + p1[3])
        return carry

      lax.fori_loop(0, CB, bag, 0)

      @pl.when(f == F - 1)
      def _():
        q = t // F
        pltpu.sync_copy(stage_v, out_hbm.at[pl.ds(b0w + q * CB, CB)])

    fire_idx(0, 0)
    fire_idx(1, 1)
    fire_idx(2, 2)
    drain_idx(0)
    addoff(0, 0)
    fire_gathers(0, 0)

    last = NSTEP - 1

    def pair4(i, carry):
      for u in range(4):
        j = 4 * i + u
        kj1 = (u + 1) % 4
        kj3 = (u + 3) % 4
        rj = u % 2
        rj1 = (u + 1) % 2
        tj1 = jnp.minimum(j + 1, last)
        tj3 = jnp.minimum(j + 3, last)
        drain_idx(kj1)
        addoff(tj1, kj1)
        fire_gathers(kj1, rj1)
        fire_idx(tj3, kj3)
        drain_gathers(rj)
        reduce_step(j, rj)
      return carry

    lax.fori_loop(0, NSTEP // 4, pair4, 0)

    drain_gathers(0)
    drain_idx(1)
    drain_idx(2)

  return kern


_kern = _make_kernel()


@jax.jit
def kernel(indices, tables):
  idx_flat = indices.reshape(F * B * L).astype(jnp.int32)
  tab_flat = tables.reshape(F * V, D)
  out = _kern(tab_flat, idx_flat)
  return out.reshape(B, F, D)
